# baseline (speedup 1.0000x reference)
import functools

import jax
import jax.numpy as jnp
from jax import lax
from jax.experimental import pallas as pl
from jax.experimental.pallas import tpu as pltpu
from jax.experimental.pallas import tpu_sc as plsc

U, W, NUM_CATS, B = 16, 16, 100000, 4096
R = U * W
NC, NS = 2, 16
NW = NC * NS
RPW = R // NW
P1 = 49920
P2 = 99968
SZ0 = P1
SZ1 = P2 - P1
TAIL = NUM_CATS - P2
NSTEP = RPW * 2


def _sc_gather(A2d, A_tail, assignment):
  mesh = plsc.VectorSubcoreMesh(core_axis_name="c", subcore_axis_name="s")

  @functools.partial(
      pl.kernel,
      out_type=jax.ShapeDtypeStruct((R, B), jnp.float32),
      mesh=mesh,
      scratch_types=[
          pltpu.VMEM((B,), jnp.int32),
          pltpu.VMEM((SZ1 + TAIL,), jnp.float32),
          pltpu.VMEM((SZ1 + TAIL,), jnp.float32),
          pltpu.VMEM((RPW, TAIL), jnp.float32),
          pltpu.VMEM((B,), jnp.float32),
          pltpu.VMEM((B,), jnp.float32),
          pltpu.SemaphoreType.DMA,
          pltpu.SemaphoreType.DMA,
          pltpu.SemaphoreType.DMA,
          pltpu.SemaphoreType.DMA,
      ],
      compiler_params=pltpu.CompilerParams(needs_layout_passes=False),
  )
  def gather_kernel(a_hbm, atail_hbm, asg_hbm, g_hbm, asg_v, buf0, buf1,
                    tail_v, grow0, grow1, sem0, sem1, wsem0, wsem1):
    wid = lax.axis_index("c") * NS + lax.axis_index("s")
    row0 = wid * RPW
    bufs = [buf0, buf1]
    sems = [sem0, sem1]
    grows = [grow0, grow1]
    wsems = [wsem0, wsem1]

    def fire(s):
      p = s % 2
      row = row0 + s // 2
      if s % 2 == 0:
        src = a_hbm.at[row].at[pl.ds(0, SZ0)]
        dst = bufs[p].at[pl.ds(0, SZ0)]
      else:
        src = a_hbm.at[row].at[pl.ds(P1, SZ1)]
        dst = bufs[p].at[pl.ds(0, SZ1)]
      return pltpu.async_copy(src, dst, sems[p])

    descs = [fire(0), fire(1)]
    pltpu.sync_copy(asg_hbm, asg_v)
    pltpu.sync_copy(atail_hbm.at[pl.ds(row0, RPW)], tail_v)
    wdescs = {}
    for s in range(NSTEP):
      descs[s].wait()
      half = s % 2
      row = s // 2
      buf = bufs[half]
      grow_v = grows[row % 2]
      if half == 0 and row >= 2:
        wdescs[row - 2].wait()
      if half == 1:
        buf[pl.ds(SZ1, 16)] = tail_v[s // 2, pl.ds(0, 16)]
        buf[pl.ds(SZ1 + 16, 16)] = tail_v[s // 2, pl.ds(16, 16)]

      def body(i, _, half=half, buf=buf, grow_v=grow_v):
        for j in range(4):
          off = i * 64 + j * 16
          a = asg_v[pl.ds(off, 16)]
          if half == 0:
            m = a < P1
            grow_v[pl.ds(off, 16)] = plsc.load_gather(buf, [a], mask=m)
          else:
            m1 = a >= P1
            g = plsc.load_gather(buf, [a - P1], mask=m1)
            grow_v[pl.ds(off, 16)] = jnp.where(m1, g, grow_v[pl.ds(off, 16)])
        return 0

      lax.fori_loop(0, B // 64, body, 0)
      if s + 2 < NSTEP:
        descs.append(fire(s + 2))
      if half == 1:
        wdescs[row] = pltpu.async_copy(
            grow_v, g_hbm.at[row0 + row], wsems[row % 2]
        )
    wdescs[RPW - 2].wait()
    wdescs[RPW - 1].wait()

  return gather_kernel(A2d, A_tail, assignment)


def _tc_expand(G3):
  UB = 4

  def body(g_ref, o_ref):
    for k in range(UB):
      g = g_ref[k]
      o_ref[k] = g[:, None, :] + g[None, :, :]

  return pl.pallas_call(
      body,
      grid=(U // UB,),
      in_specs=[pl.BlockSpec((UB, W, B), lambda u: (u, 0, 0))],
      out_specs=pl.BlockSpec((UB, W, W, B), lambda u: (u, 0, 0, 0)),
      out_shape=jax.ShapeDtypeStruct((U, W, W, B), jnp.float32),
  )(G3)


@jax.jit
def kernel(A, assignment):
  A2d = A.reshape(R, NUM_CATS)
  A_tail = A2d[:, P2:]
  G = _sc_gather(A2d, A_tail, assignment)
  return _tc_expand(G.reshape(U, W, B))


# speedup vs baseline: 1.0210x; 1.0210x over previous
import functools

import jax
import jax.numpy as jnp
from jax import lax
from jax.experimental import pallas as pl
from jax.experimental.pallas import tpu as pltpu
from jax.experimental.pallas import tpu_sc as plsc

U, W, NUM_CATS, B = 16, 16, 100000, 4096
R = U * W
NC, NS = 2, 16
NW = NC * NS
RPW = R // NW
P1 = 49920
P2 = 99968
SZ0 = P1
SZ1 = P2 - P1
TAIL = NUM_CATS - P2
NSTEP = RPW * 2


def _sc_gather(A2d, A_tail, assignment):
  mesh = plsc.VectorSubcoreMesh(core_axis_name="c", subcore_axis_name="s")

  @functools.partial(
      pl.kernel,
      out_type=jax.ShapeDtypeStruct((R, B), jnp.float32),
      mesh=mesh,
      scratch_types=[
          pltpu.VMEM((B,), jnp.int32),
          pltpu.VMEM((SZ1 + TAIL,), jnp.float32),
          pltpu.VMEM((SZ1 + TAIL,), jnp.float32),
          pltpu.VMEM((RPW, TAIL), jnp.float32),
          pltpu.VMEM((B,), jnp.float32),
          pltpu.VMEM((B,), jnp.float32),
          pltpu.SemaphoreType.DMA,
          pltpu.SemaphoreType.DMA,
          pltpu.SemaphoreType.DMA,
          pltpu.SemaphoreType.DMA,
      ],
      compiler_params=pltpu.CompilerParams(needs_layout_passes=False),
  )
  def gather_kernel(a_hbm, atail_hbm, asg_hbm, g_hbm, asg_v, buf0, buf1,
                    tail_v, grow0, grow1, sem0, sem1, wsem0, wsem1):
    wid = lax.axis_index("c") * NS + lax.axis_index("s")
    row0 = wid * RPW
    bufs = [buf0, buf1]
    sems = [sem0, sem1]
    grows = [grow0, grow1]
    wsems = [wsem0, wsem1]

    def fire(s):
      p = s % 2
      row = row0 + s // 2
      if s % 2 == 0:
        src = a_hbm.at[row].at[pl.ds(0, SZ0)]
        dst = bufs[p].at[pl.ds(0, SZ0)]
      else:
        src = a_hbm.at[row].at[pl.ds(P1, SZ1)]
        dst = bufs[p].at[pl.ds(0, SZ1)]
      return pltpu.async_copy(src, dst, sems[p])

    descs = [fire(0), fire(1)]
    pltpu.sync_copy(asg_hbm, asg_v)
    pltpu.sync_copy(atail_hbm.at[pl.ds(row0, RPW)], tail_v)
    wdescs = {}
    for s in range(NSTEP):
      descs[s].wait()
      half = s % 2
      row = s // 2
      buf = bufs[half]
      grow_v = grows[row % 2]
      if half == 0 and row >= 2:
        wdescs[row - 2].wait()
      if half == 1:
        buf[pl.ds(SZ1, 16)] = tail_v[s // 2, pl.ds(0, 16)]
        buf[pl.ds(SZ1 + 16, 16)] = tail_v[s // 2, pl.ds(16, 16)]

      def body(i, _, half=half, buf=buf, grow_v=grow_v):
        for j in range(4):
          off = i * 64 + j * 16
          a = asg_v[pl.ds(off, 16)]
          if half == 0:
            m = a < P1
            grow_v[pl.ds(off, 16)] = plsc.load_gather(buf, [a], mask=m)
          else:
            m1 = a >= P1
            g = plsc.load_gather(buf, [a - P1], mask=m1)
            grow_v[pl.ds(off, 16)] = jnp.where(m1, g, grow_v[pl.ds(off, 16)])
        return 0

      lax.fori_loop(0, B // 64, body, 0)
      if s + 2 < NSTEP:
        descs.append(fire(s + 2))
      if half == 1:
        wdescs[row] = pltpu.async_copy(
            grow_v, g_hbm.at[row0 + row], wsems[row % 2]
        )
    wdescs[RPW - 2].wait()
    wdescs[RPW - 1].wait()

  return gather_kernel(A2d, A_tail, assignment)


def _tc_expand(G3):
  UB = 2

  def body(g_ref, o_ref):
    for k in range(UB):
      g = g_ref[k]
      o_ref[k] = g[:, None, :] + g[None, :, :]

  return pl.pallas_call(
      body,
      grid=(U // UB,),
      in_specs=[pl.BlockSpec((UB, W, B), lambda u: (u, 0, 0))],
      out_specs=pl.BlockSpec((UB, W, W, B), lambda u: (u, 0, 0, 0)),
      out_shape=jax.ShapeDtypeStruct((U, W, W, B), jnp.float32),
  )(G3)


@jax.jit
def kernel(A, assignment):
  A2d = A.reshape(R, NUM_CATS)
  A_tail = A2d[:, P2:]
  G = _sc_gather(A2d, A_tail, assignment)
  return _tc_expand(G.reshape(U, W, B))
